# baseline (speedup 1.0000x reference)
import jax
import jax.numpy as jnp
from jax import lax
from jax.experimental import pallas as pl
from jax.experimental.pallas import tpu as pltpu
from jax.experimental.pallas import tpu_sc as plsc

N = 10000
E = 320000
D = 128
H = 128
C = 40

NPAD = 10240
NC, NS = 2, 16
CH = 88
CHUNKS = 228
EPT = CHUNKS * CH
EPAD = EPT * NS
RPT = NPAD // NS
NBUF = 4
LOOK = 1

_F32 = jnp.float32


def _make_seg_kernel(compute_deg):
  out_type = [
      jax.ShapeDtypeStruct((NPAD, D), _F32),
      jax.ShapeDtypeStruct((NPAD, D), _F32),
  ]
  if compute_deg:
    out_type += [
        jax.ShapeDtypeStruct((NPAD,), _F32),
        jax.ShapeDtypeStruct((NPAD,), _F32),
    ]
  mesh = plsc.VectorSubcoreMesh(
      core_axis_name="c", subcore_axis_name="s", num_cores=NC, num_subcores=NS)

  scratch = [
      [pltpu.VMEM((CH,), jnp.int32) for _ in range(NBUF)],
      [pltpu.VMEM((CH,), jnp.int32) for _ in range(NBUF)],
      [pltpu.VMEM((CH, D), _F32) for _ in range(NBUF)],
      pltpu.VMEM((RPT,), _F32),
      pltpu.VMEM((96,), _F32),
      pltpu.VMEM_SHARED((NPAD, D), _F32),
      pltpu.VMEM_SHARED((NPAD,), _F32),
      [pltpu.SemaphoreType.DMA for _ in range(NBUF)],
      [pltpu.SemaphoreType.DMA for _ in range(NBUF)],
      [pltpu.SemaphoreType.DMA for _ in range(NBUF)],
  ]

  def body(x0, x1, src0, dst0, src1, dst1, *rest):
    if compute_deg:
      agg0, agg1, deg0, deg1 = rest[:4]
      scr = rest[4:]
    else:
      agg0, agg1 = rest[:2]
      deg0 = deg1 = None
      scr = rest[2:]
    (src_i, dst_i, rows, degbuf, ones_v, acc_sh, deg_sh, gsem, ssem,
     dsem) = scr
    stage = rows[0]

    wid = lax.axis_index("s")
    cid = lax.axis_index("c")
    r0 = wid * RPT

    def zrow(i, _):
      def zcol(j, _):
        stage[i, pl.ds(j * 16, 16)] = jnp.zeros((16,), _F32)
        return 0
      return lax.fori_loop(0, D // 16, zcol, 0)
    lax.fori_loop(0, CH, zrow, 0)

    if compute_deg:
      def zdeg(j, _):
        degbuf[pl.ds(j * 16, 16)] = jnp.zeros((16,), _F32)
        return 0
      lax.fori_loop(0, RPT // 16, zdeg, 0)

      def fones(j, _):
        ones_v[pl.ds(j * 16, 16)] = jnp.ones((16,), _F32)
        return 0
      lax.fori_loop(0, 96 // 16, fones, 0)
      pltpu.sync_copy(degbuf, deg_sh.at[pl.ds(r0, RPT)])

    ZB = RPT // CH + 1
    for j in range(ZB):
      take = min(CH, RPT - j * CH)
      pltpu.sync_copy(stage.at[pl.ds(0, take)],
                      acc_sh.at[pl.ds(r0 + j * CH, take)])

    plsc.subcore_barrier()

    def run(x_hbm, src_hbm, dst_hbm):
      def load_idx(c, b):
        pltpu.sync_copy(src_hbm.at[wid, c], src_i[b])
        pltpu.sync_copy(dst_hbm.at[wid, c], dst_i[b])

      def gather_start(b):
        pltpu.async_copy(x_hbm.at[src_i[b]], rows[b], gsem[b])

      def gather_wait(b):
        pltpu.make_async_copy(x_hbm.at[src_i[b]], rows[b], gsem[b]).wait()

      def scatter_start(b):
        pltpu.async_copy(rows[b], acc_sh.at[dst_i[b]], ssem[b], add=True)

      def scatter_wait(b):
        pltpu.make_async_copy(rows[b], acc_sh.at[dst_i[b]], ssem[b]).wait()

      for b in range(NBUF):
        load_idx(b, b)
        gather_start(b)

      n_groups = (CHUNKS + LOOK + NBUF - 1) // NBUF

      def group(g, _):
        for b in range(NBUF):
          it = g * NBUF + b
          jb = (b - LOOK) % NBUF

          @pl.when(it < CHUNKS)
          def _():
            gather_wait(b)
            scatter_start(b)
            if compute_deg:
              pltpu.async_copy(ones_v.at[pl.ds(0, CH)], deg_sh.at[dst_i[b]],
                               dsem[b], add=True)

          @pl.when(jnp.logical_and(it >= LOOK, it < CHUNKS + LOOK))
          def _():
            scatter_wait(jb)
            if compute_deg:
              pltpu.make_async_copy(ones_v.at[pl.ds(0, CH)],
                                    deg_sh.at[dst_i[jb]], dsem[jb]).wait()
            nxt = it - LOOK + NBUF

            @pl.when(nxt < CHUNKS)
            def _():
              load_idx(nxt, jb)
              gather_start(jb)
        return 0
      lax.fori_loop(0, n_groups, group, 0)

    @pl.when(cid == 0)
    def _():
      run(x0, src0, dst0)

    @pl.when(cid == 1)
    def _():
      run(x1, src1, dst1)

    plsc.subcore_barrier()

    def copy_out(agg_out, deg_out):
      for j in range(ZB):
        take = min(CH, RPT - j * CH)
        pltpu.sync_copy(acc_sh.at[pl.ds(r0 + j * CH, take)],
                        stage.at[pl.ds(0, take)])
        pltpu.sync_copy(stage.at[pl.ds(0, take)],
                        agg_out.at[pl.ds(r0 + j * CH, take)])
      if compute_deg:
        pltpu.sync_copy(deg_sh.at[pl.ds(r0, RPT)], degbuf)
        pltpu.sync_copy(degbuf, deg_out.at[pl.ds(r0, RPT)])

    @pl.when(cid == 0)
    def _():
      copy_out(agg0, deg0)

    @pl.when(cid == 1)
    def _():
      copy_out(agg1, deg1)

  return pl.kernel(body, out_type=out_type, mesh=mesh, scratch_types=scratch)


_seg_with_deg = _make_seg_kernel(True)
_seg_no_deg = _make_seg_kernel(False)


BR = 1024


def _mm(a, b):
  return jnp.dot(a.astype(jnp.bfloat16), b.astype(jnp.bfloat16),
                 preferred_element_type=_F32)


def _layer0_body(feat, aL, aG, dG, dK, wls, wln, bl, wgs, wgn, bg, hl, hg):
  x = feat[...]
  mL = aL[...] / jnp.maximum(dG[...], 1.0)
  mG = aG[...] / jnp.maximum(dK[...], 1.0)
  hl[...] = jnp.maximum(_mm(x, wls[...]) + _mm(mL, wln[...]) + bl[...], 0.0)
  hg[...] = jnp.maximum(_mm(x, wgs[...]) + _mm(mG, wgn[...]) + bg[...], 0.0)


def _layer1_mlp_body(hl, aL, hg, aG, dG, dK,
                     wls, wln, bl, wgs, wgn, bg,
                     m0a, m0b, m0bias, scale, beta, m1w, m1bias, out):
  mL = aL[...] / jnp.maximum(dG[...], 1.0)
  mG = aG[...] / jnp.maximum(dK[...], 1.0)
  loc = _mm(hl[...], wls[...]) + _mm(mL, wln[...]) + bl[...]
  glo = _mm(hg[...], wgs[...]) + _mm(mG, wgn[...]) + bg[...]
  x = _mm(loc, m0a[...]) + _mm(glo, m0b[...]) + m0bias[...]
  x = jnp.maximum(x * scale[...] + beta[...], 0.0)
  out[...] = _mm(x, m1w[...]) + m1bias[...]


def _row_spec(cols):
  return pl.BlockSpec((BR, cols), lambda i: (i, 0))


def _full_spec(r, c):
  return pl.BlockSpec((r, c), lambda i: (0, 0))


_GRID = (NPAD // BR,)
_TC_PARAMS = pltpu.CompilerParams(dimension_semantics=("parallel",))

_layer0_call = pl.pallas_call(
    _layer0_body,
    grid=_GRID,
    in_specs=[_row_spec(D), _row_spec(D), _row_spec(D),
              _row_spec(1), _row_spec(1),
              _full_spec(D, H), _full_spec(D, H), _full_spec(1, H),
              _full_spec(D, H), _full_spec(D, H), _full_spec(1, H)],
    out_specs=[_row_spec(H), _row_spec(H)],
    out_shape=[jax.ShapeDtypeStruct((NPAD, H), _F32),
               jax.ShapeDtypeStruct((NPAD, H), _F32)],
    compiler_params=_TC_PARAMS,
)

_layer1_call = pl.pallas_call(
    _layer1_mlp_body,
    grid=_GRID,
    in_specs=[_row_spec(H), _row_spec(H), _row_spec(H), _row_spec(H),
              _row_spec(1), _row_spec(1),
              _full_spec(H, H), _full_spec(H, H), _full_spec(1, H),
              _full_spec(H, H), _full_spec(H, H), _full_spec(1, H),
              _full_spec(H, H // 2), _full_spec(H, H // 2),
              _full_spec(1, H // 2), _full_spec(1, H // 2),
              _full_spec(1, H // 2),
              _full_spec(H // 2, C), _full_spec(1, C)],
    out_specs=[_row_spec(C)],
    out_shape=[jax.ShapeDtypeStruct((NPAD, C), _F32)],
    compiler_params=_TC_PARAMS,
)


def kernel(feat, g_edge_index, knn_edge_index,
           l0_Ws, l0_Wn, l0_b, l1_Ws, l1_Wn, l1_b,
           g0_Ws, g0_Wn, g0_b, g1_Ws, g1_Wn, g1_b,
           m0_W, m0_b, bn_gamma, bn_beta, m1_W, m1_b):
  feat_p = jnp.pad(feat, ((0, NPAD - N), (0, 0)))

  def prep_edges(ei):
    src = jnp.pad(ei[0], (0, EPAD - E))
    dst = jnp.pad(ei[1], (0, EPAD - E), constant_values=N)
    return src.reshape(NS, CHUNKS, CH), dst.reshape(NS, CHUNKS, CH)

  gs, gd = prep_edges(g_edge_index)
  ks, kd = prep_edges(knn_edge_index)

  aggL0, aggG0, degG, degK = _seg_with_deg(feat_p, feat_p, gs, gd, ks, kd)
  dG = degG.reshape(NPAD, 1)
  dK = degK.reshape(NPAD, 1)

  hl0, hg0 = _layer0_call(
      feat_p, aggL0, aggG0, dG, dK,
      l0_Ws, l0_Wn, l0_b.reshape(1, H),
      g0_Ws, g0_Wn, g0_b.reshape(1, H))

  aggL1, aggG1 = _seg_no_deg(hl0, hg0, gs, gd, ks, kd)

  scale = (bn_gamma / jnp.sqrt(1.0 + 1e-5)).reshape(1, H // 2)
  (out,) = _layer1_call(
      hl0, aggL1, hg0, aggG1, dG, dK,
      l1_Ws, l1_Wn, l1_b.reshape(1, H),
      g1_Ws, g1_Wn, g1_b.reshape(1, H),
      m0_W[:H], m0_W[H:], m0_b.reshape(1, H // 2),
      scale, bn_beta.reshape(1, H // 2),
      m1_W, m1_b.reshape(1, C))
  return out[:N]


# speedup vs baseline: 1.0019x; 1.0019x over previous
import jax
import jax.numpy as jnp
from jax import lax
from jax.experimental import pallas as pl
from jax.experimental.pallas import tpu as pltpu
from jax.experimental.pallas import tpu_sc as plsc

N = 10000
E = 320000
D = 128
H = 128
C = 40

NPAD = 10240
NC, NS = 2, 16
CH = 88
CHUNKS = 228
EPT = CHUNKS * CH
EPAD = EPT * NS
RPT = NPAD // NS
NBUF = 4
LOOK = 1

_F32 = jnp.float32


def _make_seg_kernel(compute_deg):
  out_type = [
      jax.ShapeDtypeStruct((NPAD, D), _F32),
      jax.ShapeDtypeStruct((NPAD, D), _F32),
  ]
  if compute_deg:
    out_type += [
        jax.ShapeDtypeStruct((NPAD,), _F32),
        jax.ShapeDtypeStruct((NPAD,), _F32),
    ]
  mesh = plsc.VectorSubcoreMesh(
      core_axis_name="c", subcore_axis_name="s", num_cores=NC, num_subcores=NS)

  scratch = [
      [pltpu.VMEM((CH,), jnp.int32) for _ in range(NBUF)],
      [pltpu.VMEM((CH,), jnp.int32) for _ in range(NBUF)],
      [pltpu.VMEM((CH, D), _F32) for _ in range(NBUF)],
      pltpu.VMEM((RPT,), _F32),
      pltpu.VMEM((96,), _F32),
      pltpu.VMEM_SHARED((NPAD, D), _F32),
      pltpu.VMEM_SHARED((NPAD,), _F32),
      [pltpu.SemaphoreType.DMA for _ in range(NBUF)],
      [pltpu.SemaphoreType.DMA for _ in range(NBUF)],
      [pltpu.SemaphoreType.DMA for _ in range(NBUF)],
  ]

  def body(x0, x1, src0, dst0, src1, dst1, *rest):
    if compute_deg:
      agg0, agg1, deg0, deg1 = rest[:4]
      scr = rest[4:]
    else:
      agg0, agg1 = rest[:2]
      deg0 = deg1 = None
      scr = rest[2:]
    (src_i, dst_i, rows, degbuf, ones_v, acc_sh, deg_sh, gsem, ssem,
     dsem) = scr
    stage = rows[0]

    wid = lax.axis_index("s")
    cid = lax.axis_index("c")
    r0 = wid * RPT

    def zrow(i, _):
      def zcol(j, _):
        stage[i, pl.ds(j * 16, 16)] = jnp.zeros((16,), _F32)
        return 0
      return lax.fori_loop(0, D // 16, zcol, 0)
    lax.fori_loop(0, CH, zrow, 0)

    if compute_deg:
      def zdeg(j, _):
        degbuf[pl.ds(j * 16, 16)] = jnp.zeros((16,), _F32)
        return 0
      lax.fori_loop(0, RPT // 16, zdeg, 0)

      def fones(j, _):
        ones_v[pl.ds(j * 16, 16)] = jnp.ones((16,), _F32)
        return 0
      lax.fori_loop(0, 96 // 16, fones, 0)
      pltpu.sync_copy(degbuf, deg_sh.at[pl.ds(r0, RPT)])

    ZB = RPT // CH + 1
    for j in range(ZB):
      take = min(CH, RPT - j * CH)
      pltpu.sync_copy(stage.at[pl.ds(0, take)],
                      acc_sh.at[pl.ds(r0 + j * CH, take)])

    plsc.subcore_barrier()

    def run(x_hbm, src_hbm, dst_hbm):
      def load_idx(c, b):
        pltpu.sync_copy(src_hbm.at[wid, c], src_i[b])
        pltpu.sync_copy(dst_hbm.at[wid, c], dst_i[b])

      def gather_start(b):
        pltpu.async_copy(x_hbm.at[src_i[b]], rows[b], gsem[b])

      def gather_wait(b):
        pltpu.make_async_copy(x_hbm.at[src_i[b]], rows[b], gsem[b]).wait()

      def scatter_start(b):
        pltpu.async_copy(rows[b], acc_sh.at[dst_i[b]], ssem[b], add=True)

      def scatter_wait(b):
        pltpu.make_async_copy(rows[b], acc_sh.at[dst_i[b]], ssem[b]).wait()

      for b in range(NBUF):
        load_idx(b, b)
        gather_start(b)

      n_groups = (CHUNKS + LOOK + NBUF - 1) // NBUF

      def group(g, _):
        for b in range(NBUF):
          it = g * NBUF + b
          jb = (b - LOOK) % NBUF

          @pl.when(it < CHUNKS)
          def _():
            gather_wait(b)
            scatter_start(b)
            if compute_deg:
              pltpu.async_copy(ones_v.at[pl.ds(0, CH)], deg_sh.at[dst_i[b]],
                               dsem[b], add=True)

          @pl.when(jnp.logical_and(it >= LOOK, it < CHUNKS + LOOK))
          def _():
            scatter_wait(jb)
            if compute_deg:
              pltpu.make_async_copy(ones_v.at[pl.ds(0, CH)],
                                    deg_sh.at[dst_i[jb]], dsem[jb]).wait()
            nxt = it - LOOK + NBUF

            @pl.when(nxt < CHUNKS)
            def _():
              load_idx(nxt, jb)
              gather_start(jb)
        return 0
      lax.fori_loop(0, n_groups, group, 0)

    @pl.when(cid == 0)
    def _():
      run(x0, src0, dst0)

    @pl.when(cid == 1)
    def _():
      run(x1, src1, dst1)

    plsc.subcore_barrier()

    def copy_out(agg_out, deg_out):
      for j in range(ZB):
        take = min(CH, RPT - j * CH)
        pltpu.async_copy(acc_sh.at[pl.ds(r0 + j * CH, take)],
                         agg_out.at[pl.ds(r0 + j * CH, take)],
                         gsem[j % NBUF])
      for j in range(ZB):
        take = min(CH, RPT - j * CH)
        pltpu.make_async_copy(acc_sh.at[pl.ds(r0 + j * CH, take)],
                              agg_out.at[pl.ds(r0 + j * CH, take)],
                              gsem[j % NBUF]).wait()
      if compute_deg:
        pltpu.sync_copy(deg_sh.at[pl.ds(r0, RPT)], degbuf)
        pltpu.sync_copy(degbuf, deg_out.at[pl.ds(r0, RPT)])

    @pl.when(cid == 0)
    def _():
      copy_out(agg0, deg0)

    @pl.when(cid == 1)
    def _():
      copy_out(agg1, deg1)

  return pl.kernel(body, out_type=out_type, mesh=mesh, scratch_types=scratch)


_seg_with_deg = _make_seg_kernel(True)
_seg_no_deg = _make_seg_kernel(False)


BR = 1024


def _mm(a, b):
  return jnp.dot(a, b, preferred_element_type=_F32)


def _layer0_body(feat, aL, aG, dG, dK, wls, wln, bl, wgs, wgn, bg, hl, hg):
  x = feat[...]
  mL = aL[...] / jnp.maximum(dG[...], 1.0)
  mG = aG[...] / jnp.maximum(dK[...], 1.0)
  hl[...] = jnp.maximum(_mm(x, wls[...]) + _mm(mL, wln[...]) + bl[...], 0.0)
  hg[...] = jnp.maximum(_mm(x, wgs[...]) + _mm(mG, wgn[...]) + bg[...], 0.0)


def _layer1_mlp_body(hl, aL, hg, aG, dG, dK,
                     wls, wln, bl, wgs, wgn, bg,
                     m0a, m0b, m0bias, scale, beta, m1w, m1bias, out):
  mL = aL[...] / jnp.maximum(dG[...], 1.0)
  mG = aG[...] / jnp.maximum(dK[...], 1.0)
  loc = _mm(hl[...], wls[...]) + _mm(mL, wln[...]) + bl[...]
  glo = _mm(hg[...], wgs[...]) + _mm(mG, wgn[...]) + bg[...]
  x = _mm(loc, m0a[...]) + _mm(glo, m0b[...]) + m0bias[...]
  x = jnp.maximum(x * scale[...] + beta[...], 0.0)
  out[...] = _mm(x, m1w[...]) + m1bias[...]


def _row_spec(cols):
  return pl.BlockSpec((BR, cols), lambda i: (i, 0))


def _full_spec(r, c):
  return pl.BlockSpec((r, c), lambda i: (0, 0))


_GRID = (NPAD // BR,)
_TC_PARAMS = pltpu.CompilerParams(dimension_semantics=("parallel",))

_layer0_call = pl.pallas_call(
    _layer0_body,
    grid=_GRID,
    in_specs=[_row_spec(D), _row_spec(D), _row_spec(D),
              _row_spec(1), _row_spec(1),
              _full_spec(D, H), _full_spec(D, H), _full_spec(1, H),
              _full_spec(D, H), _full_spec(D, H), _full_spec(1, H)],
    out_specs=[_row_spec(H), _row_spec(H)],
    out_shape=[jax.ShapeDtypeStruct((NPAD, H), _F32),
               jax.ShapeDtypeStruct((NPAD, H), _F32)],
    compiler_params=_TC_PARAMS,
)

_layer1_call = pl.pallas_call(
    _layer1_mlp_body,
    grid=_GRID,
    in_specs=[_row_spec(H), _row_spec(H), _row_spec(H), _row_spec(H),
              _row_spec(1), _row_spec(1),
              _full_spec(H, H), _full_spec(H, H), _full_spec(1, H),
              _full_spec(H, H), _full_spec(H, H), _full_spec(1, H),
              _full_spec(H, H // 2), _full_spec(H, H // 2),
              _full_spec(1, H // 2), _full_spec(1, H // 2),
              _full_spec(1, H // 2),
              _full_spec(H // 2, C), _full_spec(1, C)],
    out_specs=[_row_spec(C)],
    out_shape=[jax.ShapeDtypeStruct((NPAD, C), _F32)],
    compiler_params=_TC_PARAMS,
)


def kernel(feat, g_edge_index, knn_edge_index,
           l0_Ws, l0_Wn, l0_b, l1_Ws, l1_Wn, l1_b,
           g0_Ws, g0_Wn, g0_b, g1_Ws, g1_Wn, g1_b,
           m0_W, m0_b, bn_gamma, bn_beta, m1_W, m1_b):
  feat_p = jnp.pad(feat, ((0, NPAD - N), (0, 0)))

  def prep_edges(ei):
    src = jnp.pad(ei[0], (0, EPAD - E))
    dst = jnp.pad(ei[1], (0, EPAD - E), constant_values=N)
    return src.reshape(NS, CHUNKS, CH), dst.reshape(NS, CHUNKS, CH)

  gs, gd = prep_edges(g_edge_index)
  ks, kd = prep_edges(knn_edge_index)

  aggL0, aggG0, degG, degK = _seg_with_deg(feat_p, feat_p, gs, gd, ks, kd)
  dG = degG.reshape(NPAD, 1)
  dK = degK.reshape(NPAD, 1)

  hl0, hg0 = _layer0_call(
      feat_p, aggL0, aggG0, dG, dK,
      l0_Ws, l0_Wn, l0_b.reshape(1, H),
      g0_Ws, g0_Wn, g0_b.reshape(1, H))

  aggL1, aggG1 = _seg_no_deg(hl0, hg0, gs, gd, ks, kd)

  scale = (bn_gamma / jnp.sqrt(1.0 + 1e-5)).reshape(1, H // 2)
  (out,) = _layer1_call(
      hl0, aggL1, hg0, aggG1, dG, dK,
      l1_Ws, l1_Wn, l1_b.reshape(1, H),
      g1_Ws, g1_Wn, g1_b.reshape(1, H),
      m0_W[:H], m0_W[H:], m0_b.reshape(1, H // 2),
      scale, bn_beta.reshape(1, H // 2),
      m1_W, m1_b.reshape(1, C))
  return out[:N]


# speedup vs baseline: 1.0916x; 1.0895x over previous
import jax
import jax.numpy as jnp
from jax import lax
from jax.experimental import pallas as pl
from jax.experimental.pallas import tpu as pltpu
from jax.experimental.pallas import tpu_sc as plsc

N = 10000
E = 320000
D = 128
H = 128
C = 40

NPAD = 10240
NC, NS = 2, 16
CH = 80
CHUNKS = 250
EPT = CHUNKS * CH
EPAD = EPT * NS
RPT = NPAD // NS
NBUF = 4
LOOK = 1

_F32 = jnp.float32


def _make_seg_kernel(compute_deg):
  out_type = [
      jax.ShapeDtypeStruct((NPAD, D), _F32),
      jax.ShapeDtypeStruct((NPAD, D), _F32),
  ]
  if compute_deg:
    out_type += [
        jax.ShapeDtypeStruct((NPAD,), _F32),
        jax.ShapeDtypeStruct((NPAD,), _F32),
    ]
  mesh = plsc.VectorSubcoreMesh(
      core_axis_name="c", subcore_axis_name="s", num_cores=NC, num_subcores=NS)

  scratch = [
      [pltpu.VMEM((CH,), jnp.int32) for _ in range(NBUF)],
      [pltpu.VMEM((CH,), jnp.int32) for _ in range(NBUF)],
      [pltpu.VMEM((CH, D), _F32) for _ in range(NBUF)],
      pltpu.VMEM((RPT,), _F32),
      pltpu.VMEM((96,), _F32),
      pltpu.VMEM_SHARED((NPAD, D), _F32),
      pltpu.VMEM_SHARED((NPAD,), _F32),
      [pltpu.SemaphoreType.DMA for _ in range(NBUF)],
      [pltpu.SemaphoreType.DMA for _ in range(NBUF)],
      [pltpu.SemaphoreType.DMA for _ in range(NBUF)],
  ]

  def body(x0, x1, src0, dst0, src1, dst1, *rest):
    if compute_deg:
      agg0, agg1, deg0, deg1 = rest[:4]
      scr = rest[4:]
    else:
      agg0, agg1 = rest[:2]
      deg0 = deg1 = None
      scr = rest[2:]
    (src_i, dst_i, rows, degbuf, ones_v, acc_sh, deg_sh, gsem, ssem,
     dsem) = scr
    stage = rows[0]

    wid = lax.axis_index("s")
    cid = lax.axis_index("c")
    r0 = wid * RPT

    def zrow(i, _):
      def zcol(j, _):
        stage[i, pl.ds(j * 16, 16)] = jnp.zeros((16,), _F32)
        return 0
      return lax.fori_loop(0, D // 16, zcol, 0)
    lax.fori_loop(0, CH, zrow, 0)

    if compute_deg:
      def zdeg(j, _):
        degbuf[pl.ds(j * 16, 16)] = jnp.zeros((16,), _F32)
        return 0
      lax.fori_loop(0, RPT // 16, zdeg, 0)

      def fones(j, _):
        ones_v[pl.ds(j * 16, 16)] = jnp.ones((16,), _F32)
        return 0
      lax.fori_loop(0, 96 // 16, fones, 0)
      pltpu.sync_copy(degbuf, deg_sh.at[pl.ds(r0, RPT)])

    ZB = RPT // CH
    for j in range(ZB):
      take = min(CH, RPT - j * CH)
      pltpu.async_copy(stage.at[pl.ds(0, take)],
                       acc_sh.at[pl.ds(r0 + j * CH, take)], ssem[j % NBUF])
    for j in range(ZB):
      take = min(CH, RPT - j * CH)
      pltpu.make_async_copy(stage.at[pl.ds(0, take)],
                            acc_sh.at[pl.ds(r0 + j * CH, take)],
                            ssem[j % NBUF]).wait()

    plsc.subcore_barrier()

    def run(x_hbm, src_hbm, dst_hbm):
      def load_idx(c, b):
        pltpu.sync_copy(src_hbm.at[wid, c], src_i[b])
        pltpu.sync_copy(dst_hbm.at[wid, c], dst_i[b])

      def gather_start(b):
        pltpu.async_copy(x_hbm.at[src_i[b]], rows[b], gsem[b])

      def gather_wait(b):
        pltpu.make_async_copy(x_hbm.at[src_i[b]], rows[b], gsem[b]).wait()

      def scatter_start(b):
        pltpu.async_copy(rows[b], acc_sh.at[dst_i[b]], ssem[b], add=True)

      def scatter_wait(b):
        pltpu.make_async_copy(rows[b], acc_sh.at[dst_i[b]], ssem[b]).wait()

      for b in range(NBUF):
        load_idx(b, b)
        gather_start(b)

      n_groups = (CHUNKS + LOOK + NBUF - 1) // NBUF

      def group(g, _):
        for b in range(NBUF):
          it = g * NBUF + b
          jb = (b - LOOK) % NBUF

          @pl.when(it < CHUNKS)
          def _():
            gather_wait(b)
            scatter_start(b)
            if compute_deg:
              pltpu.async_copy(ones_v.at[pl.ds(0, CH)], deg_sh.at[dst_i[b]],
                               dsem[b], add=True)

          @pl.when(jnp.logical_and(it >= LOOK, it < CHUNKS + LOOK))
          def _():
            scatter_wait(jb)
            if compute_deg:
              pltpu.make_async_copy(ones_v.at[pl.ds(0, CH)],
                                    deg_sh.at[dst_i[jb]], dsem[jb]).wait()
            nxt = it - LOOK + NBUF

            @pl.when(nxt < CHUNKS)
            def _():
              load_idx(nxt, jb)
              gather_start(jb)
        return 0
      lax.fori_loop(0, n_groups, group, 0)

    @pl.when(cid == 0)
    def _():
      run(x0, src0, dst0)

    @pl.when(cid == 1)
    def _():
      run(x1, src1, dst1)

    plsc.subcore_barrier()

    def copy_out(agg_out, deg_out):
      for j in range(ZB):
        take = min(CH, RPT - j * CH)
        pltpu.async_copy(acc_sh.at[pl.ds(r0 + j * CH, take)],
                         agg_out.at[pl.ds(r0 + j * CH, take)],
                         gsem[j % NBUF])
      for j in range(ZB):
        take = min(CH, RPT - j * CH)
        pltpu.make_async_copy(acc_sh.at[pl.ds(r0 + j * CH, take)],
                              agg_out.at[pl.ds(r0 + j * CH, take)],
                              gsem[j % NBUF]).wait()
      if compute_deg:
        pltpu.sync_copy(deg_sh.at[pl.ds(r0, RPT)], degbuf)
        pltpu.sync_copy(degbuf, deg_out.at[pl.ds(r0, RPT)])

    @pl.when(cid == 0)
    def _():
      copy_out(agg0, deg0)

    @pl.when(cid == 1)
    def _():
      copy_out(agg1, deg1)

  return pl.kernel(body, out_type=out_type, mesh=mesh, scratch_types=scratch)


_seg_with_deg = _make_seg_kernel(True)
_seg_no_deg = _make_seg_kernel(False)


BR = 1024


def _mm(a, b):
  return jnp.dot(a, b, preferred_element_type=_F32)


def _layer0_body(feat, aL, aG, dG, dK, wls, wln, bl, wgs, wgn, bg, hl, hg):
  x = feat[...]
  mL = aL[...] / jnp.maximum(dG[...], 1.0)
  mG = aG[...] / jnp.maximum(dK[...], 1.0)
  hl[...] = jnp.maximum(_mm(x, wls[...]) + _mm(mL, wln[...]) + bl[...], 0.0)
  hg[...] = jnp.maximum(_mm(x, wgs[...]) + _mm(mG, wgn[...]) + bg[...], 0.0)


def _layer1_mlp_body(hl, aL, hg, aG, dG, dK,
                     wls, wln, bl, wgs, wgn, bg,
                     m0a, m0b, m0bias, scale, beta, m1w, m1bias, out):
  mL = aL[...] / jnp.maximum(dG[...], 1.0)
  mG = aG[...] / jnp.maximum(dK[...], 1.0)
  loc = _mm(hl[...], wls[...]) + _mm(mL, wln[...]) + bl[...]
  glo = _mm(hg[...], wgs[...]) + _mm(mG, wgn[...]) + bg[...]
  x = _mm(loc, m0a[...]) + _mm(glo, m0b[...]) + m0bias[...]
  x = jnp.maximum(x * scale[...] + beta[...], 0.0)
  out[...] = _mm(x, m1w[...]) + m1bias[...]


def _row_spec(cols):
  return pl.BlockSpec((BR, cols), lambda i: (i, 0))


def _full_spec(r, c):
  return pl.BlockSpec((r, c), lambda i: (0, 0))


_GRID = (NPAD // BR,)
_TC_PARAMS = pltpu.CompilerParams(dimension_semantics=("parallel",))

_layer0_call = pl.pallas_call(
    _layer0_body,
    grid=_GRID,
    in_specs=[_row_spec(D), _row_spec(D), _row_spec(D),
              _row_spec(1), _row_spec(1),
              _full_spec(D, H), _full_spec(D, H), _full_spec(1, H),
              _full_spec(D, H), _full_spec(D, H), _full_spec(1, H)],
    out_specs=[_row_spec(H), _row_spec(H)],
    out_shape=[jax.ShapeDtypeStruct((NPAD, H), _F32),
               jax.ShapeDtypeStruct((NPAD, H), _F32)],
    compiler_params=_TC_PARAMS,
)

_layer1_call = pl.pallas_call(
    _layer1_mlp_body,
    grid=_GRID,
    in_specs=[_row_spec(H), _row_spec(H), _row_spec(H), _row_spec(H),
              _row_spec(1), _row_spec(1),
              _full_spec(H, H), _full_spec(H, H), _full_spec(1, H),
              _full_spec(H, H), _full_spec(H, H), _full_spec(1, H),
              _full_spec(H, H // 2), _full_spec(H, H // 2),
              _full_spec(1, H // 2), _full_spec(1, H // 2),
              _full_spec(1, H // 2),
              _full_spec(H // 2, C), _full_spec(1, C)],
    out_specs=[_row_spec(C)],
    out_shape=[jax.ShapeDtypeStruct((NPAD, C), _F32)],
    compiler_params=_TC_PARAMS,
)


def kernel(feat, g_edge_index, knn_edge_index,
           l0_Ws, l0_Wn, l0_b, l1_Ws, l1_Wn, l1_b,
           g0_Ws, g0_Wn, g0_b, g1_Ws, g1_Wn, g1_b,
           m0_W, m0_b, bn_gamma, bn_beta, m1_W, m1_b):
  feat_p = jnp.pad(feat, ((0, NPAD - N), (0, 0)))

  def prep_edges(ei):
    return (ei[0].reshape(NS, CHUNKS, CH), ei[1].reshape(NS, CHUNKS, CH))

  gs, gd = prep_edges(g_edge_index)
  ks, kd = prep_edges(knn_edge_index)

  aggL0, aggG0, degG, degK = _seg_with_deg(feat_p, feat_p, gs, gd, ks, kd)
  dG = degG.reshape(NPAD, 1)
  dK = degK.reshape(NPAD, 1)

  hl0, hg0 = _layer0_call(
      feat_p, aggL0, aggG0, dG, dK,
      l0_Ws, l0_Wn, l0_b.reshape(1, H),
      g0_Ws, g0_Wn, g0_b.reshape(1, H))

  aggL1, aggG1 = _seg_no_deg(hl0, hg0, gs, gd, ks, kd)

  scale = (bn_gamma / jnp.sqrt(1.0 + 1e-5)).reshape(1, H // 2)
  (out,) = _layer1_call(
      hl0, aggL1, hg0, aggG1, dG, dK,
      l1_Ws, l1_Wn, l1_b.reshape(1, H),
      g1_Ws, g1_Wn, g1_b.reshape(1, H),
      m0_W[:H], m0_W[H:], m0_b.reshape(1, H // 2),
      scale, bn_beta.reshape(1, H // 2),
      m1_W, m1_b.reshape(1, C))
  return out[:N]
